# baseline (device time: 355044 ns/iter reference)
import jax
import jax.numpy as jnp
from jax import lax
from jax.experimental import pallas as pl
from jax.experimental.pallas import tpu as pltpu

N_DEV = 4
M_BLK = 1024
N_TOT = 8192
HALF = N_TOT // 2
NC = 256
NCH = HALF // NC
SEED_SUB = NCH // 2


def kernel(x, w_mat):

    def body(x_hbm, w_hbm, out_hbm, comm_ref, pacc_ref, xbuf, wbuf, obuf,
             amax_ref, sendR, recvR, sendL, recvL, s1send, s1recv,
             a_send_sems, a_recv_sems, xsems, wsems, odma_sems):
        p = lax.axis_index("i")
        left = lax.rem(p + N_DEV - 1, N_DEV)
        right = lax.rem(p + 1, N_DEV)

        barrier_sem = pltpu.get_barrier_semaphore()
        for nbr in (left, right):
            pl.semaphore_signal(barrier_sem, inc=1, device_id=(nbr,),
                                device_id_type=pl.DeviceIdType.MESH)
        pl.semaphore_wait(barrier_sem, 2)

        def dir_partials(j, d, dst_ref, c0=0, c1=NCH, load_x=True,
                         xslot=None):
            base = d * HALF
            if xslot is None:
                xslot = d
            if load_x:
                xcp = pltpu.make_async_copy(
                    x_hbm.at[pl.ds(j * M_BLK, M_BLK), :], xbuf.at[xslot],
                    xsems.at[xslot])
                xcp.start()
            pltpu.make_async_copy(
                w_hbm.at[:, pl.ds(base + c0 * NC, NC)],
                wbuf.at[c0 % 2], wsems.at[c0 % 2]).start()
            if load_x:
                xcp.wait()
            x_bf = xbuf[xslot].astype(jnp.bfloat16)

            def chunk(c, _):
                buf = lax.rem(c, 2)
                nbuf = lax.rem(c + 1, 2)

                @pl.when(c + 1 < c1)
                def _():
                    pltpu.make_async_copy(
                        w_hbm.at[:, pl.ds(base + (c + 1) * NC, NC)],
                        wbuf.at[nbuf], wsems.at[nbuf]).start()

                pltpu.make_async_copy(
                    w_hbm.at[:, pl.ds(base + c * NC, NC)],
                    wbuf.at[buf], wsems.at[buf]).wait()
                acc = lax.dot_general(
                    x_bf, wbuf[buf].astype(jnp.bfloat16),
                    (((1,), (0,)), ((), ())),
                    preferred_element_type=jnp.float32)
                dst_ref[d, :, pl.ds(c * NC, NC)] = acc.astype(jnp.bfloat16)
                return 0

            lax.fori_loop(c0, c1, chunk, 0)

        def hop1_sub(d, sub, nbr):
            i = d * 2 + sub
            cols = pl.ds(sub * SEED_SUB * NC, SEED_SUB * NC)
            return pltpu.make_async_remote_copy(
                src_ref=comm_ref.at[0, d, :, cols],
                dst_ref=comm_ref.at[1, d, :, cols],
                send_sem=s1send.at[i],
                recv_sem=s1recv.at[i],
                device_id=(nbr,),
                device_id_type=pl.DeviceIdType.MESH,
            )

        def hop_full(s, d, nbr):
            send_slot = (s - 1) % 2
            recv_slot = s % 2
            ssem, rsem = (sendR, recvR) if d == 0 else (sendL, recvL)
            return pltpu.make_async_remote_copy(
                src_ref=comm_ref.at[send_slot, d],
                dst_ref=comm_ref.at[recv_slot, d],
                send_sem=ssem.at[send_slot],
                recv_sem=rsem.at[recv_slot],
                device_id=(nbr,),
                device_id_type=pl.DeviceIdType.MESH,
            )

        hop1 = {}
        dir_partials(left, 0, comm_ref.at[0], 0, SEED_SUB)
        hop1[(0, 0)] = hop1_sub(0, 0, right)
        hop1[(0, 0)].start()
        dir_partials(right, 1, comm_ref.at[0], 0, SEED_SUB)
        hop1[(1, 0)] = hop1_sub(1, 0, left)
        hop1[(1, 0)].start()
        dir_partials(left, 0, comm_ref.at[0], SEED_SUB, NCH, load_x=False)
        hop1[(0, 1)] = hop1_sub(0, 1, right)
        hop1[(0, 1)].start()
        dir_partials(right, 1, comm_ref.at[0], SEED_SUB, NCH, load_x=False)
        hop1[(1, 1)] = hop1_sub(1, 1, left)
        hop1[(1, 1)].start()

        def add_dir(d, recv_slot):
            def add_chunk(c, _):
                csl = pl.ds(c * NC, NC)
                acc = (comm_ref[recv_slot, d, :, csl].astype(jnp.float32)
                       + pacc_ref[d, :, csl].astype(jnp.float32))
                comm_ref[recv_slot, d, :, csl] = acc.astype(jnp.bfloat16)
                return 0
            lax.fori_loop(0, NCH, add_chunk, 0)

        amax = jnp.float32(0.0)
        nxt = {}
        for s in range(1, N_DEV):
            recv_slot = s % 2
            jr = lax.rem(p + 2 * N_DEV - 1 - s, N_DEV)
            jl = lax.rem(p + 1 + s, N_DEV)
            dir_partials(jr, 0, pacc_ref)
            if s % 2 == 1:
                dir_partials(jl, 1, pacc_ref, load_x=False, xslot=0)
            else:
                dir_partials(jl, 1, pacc_ref)

            if s == 1:
                waitsR = [hop1[(0, 0)], hop1[(0, 1)]]
                waitsL = [hop1[(1, 0)], hop1[(1, 1)]]
            else:
                waitsR = [nxt[0]]
                waitsL = [nxt[1]]

            if s < N_DEV - 1:
                for d, waits, nbr in ((0, waitsR, right), (1, waitsL, left)):
                    for rd in waits:
                        rd.wait()
                    add_dir(d, recv_slot)
                    nxt[d] = hop_full(s + 1, d, nbr)
                    nxt[d].start()
            else:
                for rd in waitsR + waitsL:
                    rd.wait()

                def fin_chunk(c, am):
                    d = c // NCH
                    csl = pl.ds(lax.rem(c, NCH) * NC, NC)
                    y = (comm_ref[recv_slot, d, :, csl].astype(jnp.float32)
                         + pacc_ref[d, :, csl].astype(jnp.float32))
                    y = jnp.maximum(y, 0.0)
                    comm_ref[recv_slot, d, :, csl] = y.astype(jnp.bfloat16)
                    return jnp.maximum(am, jnp.max(y))
                amax = lax.fori_loop(0, 2 * NCH, fin_chunk, amax)
        final_slot = (N_DEV - 1) % 2

        amax_ref[pl.ds(p, 1), :] = jnp.full((1, 128), amax, jnp.float32)
        sends = []
        for off in range(1, N_DEV):
            tgt = lax.rem(p + off, N_DEV)
            a = pltpu.make_async_remote_copy(
                src_ref=amax_ref.at[pl.ds(p, 1)],
                dst_ref=amax_ref.at[pl.ds(p, 1)],
                send_sem=a_send_sems.at[off - 1],
                recv_sem=a_recv_sems.at[off - 1],
                device_id=(tgt,),
                device_id_type=pl.DeviceIdType.MESH,
            )
            a.start()
            sends.append(a)
        for off in range(1, N_DEV):
            src = lax.rem(p + N_DEV - off, N_DEV)
            r = pltpu.make_async_remote_copy(
                src_ref=amax_ref.at[pl.ds(p, 1)],
                dst_ref=amax_ref.at[pl.ds(src, 1)],
                send_sem=a_send_sems.at[off - 1],
                recv_sem=a_recv_sems.at[off - 1],
                device_id=(src,),
                device_id_type=pl.DeviceIdType.MESH,
            )
            r.wait_recv()
        for a in sends:
            a.wait_send()

        amax_g = jnp.max(amax_ref[:, :])
        scale = amax_g / 127.0
        inv_scale = 127.0 / amax_g

        cps = []
        for c in range(2 * NCH):
            d = c // NCH
            b = c % 2
            csl = pl.ds((c % NCH) * NC, NC)
            if c >= 2:
                cps[c - 2].wait()
            y = comm_ref[final_slot, d, :, csl].astype(jnp.float32)
            q = jnp.clip(jnp.round(y * inv_scale), 0.0, 127.0)
            obuf[b] = (q * scale).astype(jnp.bfloat16)
            cp = pltpu.make_async_copy(
                obuf.at[b],
                out_hbm.at[:, pl.ds(d * HALF + (c % NCH) * NC, NC)],
                odma_sems.at[b])
            cp.start()
            cps.append(cp)
        cps[-2].wait()
        cps[-1].wait()

    return pl.pallas_call(
        body,
        out_shape=jax.ShapeDtypeStruct((M_BLK, N_TOT), jnp.bfloat16),
        in_specs=[
            pl.BlockSpec(memory_space=pl.ANY),
            pl.BlockSpec(memory_space=pl.ANY),
        ],
        out_specs=pl.BlockSpec(memory_space=pl.ANY),
        scratch_shapes=[
            pltpu.VMEM((2, 2, M_BLK, HALF), jnp.bfloat16),
            pltpu.VMEM((2, M_BLK, HALF), jnp.bfloat16),
            pltpu.VMEM((2, M_BLK, 1024), jnp.float32),
            pltpu.VMEM((2, 1024, NC), jnp.float32),
            pltpu.VMEM((2, M_BLK, NC), jnp.bfloat16),
            pltpu.VMEM((N_DEV, 128), jnp.float32),
            pltpu.SemaphoreType.DMA((2,)),
            pltpu.SemaphoreType.DMA((2,)),
            pltpu.SemaphoreType.DMA((2,)),
            pltpu.SemaphoreType.DMA((2,)),
            pltpu.SemaphoreType.DMA((4,)),
            pltpu.SemaphoreType.DMA((4,)),
            pltpu.SemaphoreType.DMA((N_DEV - 1,)),
            pltpu.SemaphoreType.DMA((N_DEV - 1,)),
            pltpu.SemaphoreType.DMA((2,)),
            pltpu.SemaphoreType.DMA((2,)),
            pltpu.SemaphoreType.DMA((2,)),
        ],
        compiler_params=pltpu.CompilerParams(
            collective_id=0,
            vmem_limit_bytes=67000000,
        ),
    )(x, w_mat)


# device time: 333183 ns/iter; 1.0656x vs baseline; 1.0656x over previous
import jax
import jax.numpy as jnp
from jax import lax
from jax.experimental import pallas as pl
from jax.experimental.pallas import tpu as pltpu

N_DEV = 4
M_BLK = 1024
N_TOT = 8192
HALF = N_TOT // 2
NC = 256
NCH = HALF // NC
SEED_SUB = NCH // 2


def kernel(x, w_mat):

    def body(x_hbm, w_hbm, out_hbm, comm_ref, pacc_ref, xbuf, wbuf, obuf,
             amax_ref, hsend, hrecv,
             a_send_sems, a_recv_sems, xsems, wsems, odma_sems):
        p = lax.axis_index("i")
        left = lax.rem(p + N_DEV - 1, N_DEV)
        right = lax.rem(p + 1, N_DEV)

        barrier_sem = pltpu.get_barrier_semaphore()
        for nbr in (left, right):
            pl.semaphore_signal(barrier_sem, inc=1, device_id=(nbr,),
                                device_id_type=pl.DeviceIdType.MESH)
        pl.semaphore_wait(barrier_sem, 2)

        def dir_partials(j, d, dst_ref, c0=0, c1=NCH, load_x=True,
                         xslot=None):
            base = d * HALF
            if xslot is None:
                xslot = d
            if load_x:
                xcp = pltpu.make_async_copy(
                    x_hbm.at[pl.ds(j * M_BLK, M_BLK), :], xbuf.at[xslot],
                    xsems.at[xslot])
                xcp.start()
            pltpu.make_async_copy(
                w_hbm.at[:, pl.ds(base + c0 * NC, NC)],
                wbuf.at[c0 % 2], wsems.at[c0 % 2]).start()
            if load_x:
                xcp.wait()
            x_bf = xbuf[xslot].astype(jnp.bfloat16)

            def chunk(c, _):
                buf = lax.rem(c, 2)
                nbuf = lax.rem(c + 1, 2)

                @pl.when(c + 1 < c1)
                def _():
                    pltpu.make_async_copy(
                        w_hbm.at[:, pl.ds(base + (c + 1) * NC, NC)],
                        wbuf.at[nbuf], wsems.at[nbuf]).start()

                pltpu.make_async_copy(
                    w_hbm.at[:, pl.ds(base + c * NC, NC)],
                    wbuf.at[buf], wsems.at[buf]).wait()
                acc = lax.dot_general(
                    x_bf, wbuf[buf].astype(jnp.bfloat16),
                    (((1,), (0,)), ((), ())),
                    preferred_element_type=jnp.float32)
                dst_ref[d, :, pl.ds(c * NC, NC)] = acc.astype(jnp.bfloat16)
                return 0

            lax.fori_loop(c0, c1, chunk, 0)

        def hop_rdma(s, d, sub, nbr):
            i = (s - 1) * 4 + d * 2 + sub
            cols = pl.ds(sub * SEED_SUB * NC, SEED_SUB * NC)
            return pltpu.make_async_remote_copy(
                src_ref=comm_ref.at[(s - 1) % 2, d, :, cols],
                dst_ref=comm_ref.at[s % 2, d, :, cols],
                send_sem=hsend.at[i],
                recv_sem=hrecv.at[i],
                device_id=(nbr,),
                device_id_type=pl.DeviceIdType.MESH,
            )

        pending = {}
        dir_partials(left, 0, comm_ref.at[0], 0, SEED_SUB)
        pending[(1, 0, 0)] = hop_rdma(1, 0, 0, right)
        pending[(1, 0, 0)].start()
        dir_partials(right, 1, comm_ref.at[0], 0, SEED_SUB)
        pending[(1, 1, 0)] = hop_rdma(1, 1, 0, left)
        pending[(1, 1, 0)].start()
        dir_partials(left, 0, comm_ref.at[0], SEED_SUB, NCH, load_x=False)
        pending[(1, 0, 1)] = hop_rdma(1, 0, 1, right)
        pending[(1, 0, 1)].start()
        dir_partials(right, 1, comm_ref.at[0], SEED_SUB, NCH, load_x=False)
        pending[(1, 1, 1)] = hop_rdma(1, 1, 1, left)
        pending[(1, 1, 1)].start()

        def add_range(d, recv_slot, c0, c1):
            def add_chunk(c, _):
                csl = pl.ds(c * NC, NC)
                acc = (comm_ref[recv_slot, d, :, csl].astype(jnp.float32)
                       + pacc_ref[d, :, csl].astype(jnp.float32))
                comm_ref[recv_slot, d, :, csl] = acc.astype(jnp.bfloat16)
                return 0
            lax.fori_loop(c0, c1, add_chunk, 0)

        def fin_range(d, recv_slot, c0, c1, am):
            def fin_chunk(c, am):
                csl = pl.ds(c * NC, NC)
                y = (comm_ref[recv_slot, d, :, csl].astype(jnp.float32)
                     + pacc_ref[d, :, csl].astype(jnp.float32))
                y = jnp.maximum(y, 0.0)
                comm_ref[recv_slot, d, :, csl] = y.astype(jnp.bfloat16)
                return jnp.maximum(am, jnp.max(y))
            return lax.fori_loop(c0, c1, fin_chunk, am)

        amax = jnp.float32(0.0)
        for s in range(1, N_DEV):
            recv_slot = s % 2
            jr = lax.rem(p + 2 * N_DEV - 1 - s, N_DEV)
            jl = lax.rem(p + 1 + s, N_DEV)
            dir_partials(jr, 0, pacc_ref)
            if s % 2 == 1:
                dir_partials(jl, 1, pacc_ref, load_x=False, xslot=0)
            else:
                dir_partials(jl, 1, pacc_ref)

            for sub in (0, 1):
                for d, nbr in ((0, right), (1, left)):
                    pending[(s, d, sub)].wait()
                    if s < N_DEV - 1:
                        add_range(d, recv_slot,
                                  sub * SEED_SUB, (sub + 1) * SEED_SUB)
                        nxt = hop_rdma(s + 1, d, sub, nbr)
                        nxt.start()
                        pending[(s + 1, d, sub)] = nxt
                    else:
                        amax = fin_range(d, recv_slot, sub * SEED_SUB,
                                         (sub + 1) * SEED_SUB, amax)
        final_slot = (N_DEV - 1) % 2

        amax_ref[pl.ds(p, 1), :] = jnp.full((1, 128), amax, jnp.float32)
        sends = []
        for off in range(1, N_DEV):
            tgt = lax.rem(p + off, N_DEV)
            a = pltpu.make_async_remote_copy(
                src_ref=amax_ref.at[pl.ds(p, 1)],
                dst_ref=amax_ref.at[pl.ds(p, 1)],
                send_sem=a_send_sems.at[off - 1],
                recv_sem=a_recv_sems.at[off - 1],
                device_id=(tgt,),
                device_id_type=pl.DeviceIdType.MESH,
            )
            a.start()
            sends.append(a)
        for off in range(1, N_DEV):
            src = lax.rem(p + N_DEV - off, N_DEV)
            r = pltpu.make_async_remote_copy(
                src_ref=amax_ref.at[pl.ds(p, 1)],
                dst_ref=amax_ref.at[pl.ds(src, 1)],
                send_sem=a_send_sems.at[off - 1],
                recv_sem=a_recv_sems.at[off - 1],
                device_id=(src,),
                device_id_type=pl.DeviceIdType.MESH,
            )
            r.wait_recv()
        for a in sends:
            a.wait_send()

        amax_g = jnp.max(amax_ref[:, :])
        scale = amax_g / 127.0
        inv_scale = 127.0 / amax_g

        cps = []
        for c in range(2 * NCH):
            d = c // NCH
            b = c % 2
            csl = pl.ds((c % NCH) * NC, NC)
            if c >= 2:
                cps[c - 2].wait()
            y = comm_ref[final_slot, d, :, csl].astype(jnp.float32)
            q = jnp.clip(jnp.round(y * inv_scale), 0.0, 127.0)
            obuf[b] = (q * scale).astype(jnp.bfloat16)
            cp = pltpu.make_async_copy(
                obuf.at[b],
                out_hbm.at[:, pl.ds(d * HALF + (c % NCH) * NC, NC)],
                odma_sems.at[b])
            cp.start()
            cps.append(cp)
        cps[-2].wait()
        cps[-1].wait()

    return pl.pallas_call(
        body,
        out_shape=jax.ShapeDtypeStruct((M_BLK, N_TOT), jnp.bfloat16),
        in_specs=[
            pl.BlockSpec(memory_space=pl.ANY),
            pl.BlockSpec(memory_space=pl.ANY),
        ],
        out_specs=pl.BlockSpec(memory_space=pl.ANY),
        scratch_shapes=[
            pltpu.VMEM((2, 2, M_BLK, HALF), jnp.bfloat16),
            pltpu.VMEM((2, M_BLK, HALF), jnp.bfloat16),
            pltpu.VMEM((2, M_BLK, 1024), jnp.float32),
            pltpu.VMEM((2, 1024, NC), jnp.float32),
            pltpu.VMEM((2, M_BLK, NC), jnp.bfloat16),
            pltpu.VMEM((N_DEV, 128), jnp.float32),
            pltpu.SemaphoreType.DMA((12,)),
            pltpu.SemaphoreType.DMA((12,)),
            pltpu.SemaphoreType.DMA((N_DEV - 1,)),
            pltpu.SemaphoreType.DMA((N_DEV - 1,)),
            pltpu.SemaphoreType.DMA((2,)),
            pltpu.SemaphoreType.DMA((2,)),
            pltpu.SemaphoreType.DMA((2,)),
        ],
        compiler_params=pltpu.CompilerParams(
            collective_id=0,
            vmem_limit_bytes=67000000,
        ),
    )(x, w_mat)


# device time: 323586 ns/iter; 1.0972x vs baseline; 1.0297x over previous
import jax
import jax.numpy as jnp
from jax import lax
from jax.experimental import pallas as pl
from jax.experimental.pallas import tpu as pltpu

N_DEV = 4
M_BLK = 1024
N_TOT = 8192
HALF = N_TOT // 2
NC = 256
NCH = HALF // NC
SEED_SUB = NCH // 2


def kernel(x, w_mat):

    def body(x_hbm, w_hbm, out_hbm, comm_ref, pacc_ref, xbuf, wbuf, obuf,
             amax_ref, hsend, hrecv,
             a_send_sems, a_recv_sems, xsems, wsems, odma_sems):
        p = lax.axis_index("i")
        left = lax.rem(p + N_DEV - 1, N_DEV)
        right = lax.rem(p + 1, N_DEV)

        def dir_partials(j, d, dst_ref, c0=0, c1=NCH, load_x=True,
                         xslot=None, skip_w_start=False):
            base = d * HALF
            if xslot is None:
                xslot = d
            if load_x:
                xcp = pltpu.make_async_copy(
                    x_hbm.at[pl.ds(j * M_BLK, M_BLK), :], xbuf.at[xslot],
                    xsems.at[xslot])
                xcp.start()
            if not skip_w_start:
                pltpu.make_async_copy(
                    w_hbm.at[:, pl.ds(base + c0 * NC, NC)],
                    wbuf.at[c0 % 2], wsems.at[c0 % 2]).start()
            if load_x:
                xcp.wait()
            x_bf = xbuf[xslot].astype(jnp.bfloat16)

            def chunk(c, _):
                buf = lax.rem(c, 2)
                nbuf = lax.rem(c + 1, 2)

                @pl.when(c + 1 < c1)
                def _():
                    pltpu.make_async_copy(
                        w_hbm.at[:, pl.ds(base + (c + 1) * NC, NC)],
                        wbuf.at[nbuf], wsems.at[nbuf]).start()

                pltpu.make_async_copy(
                    w_hbm.at[:, pl.ds(base + c * NC, NC)],
                    wbuf.at[buf], wsems.at[buf]).wait()
                acc = lax.dot_general(
                    x_bf, wbuf[buf].astype(jnp.bfloat16),
                    (((1,), (0,)), ((), ())),
                    preferred_element_type=jnp.float32)
                dst_ref[d, :, pl.ds(c * NC, NC)] = acc.astype(jnp.bfloat16)
                return 0

            lax.fori_loop(c0, c1, chunk, 0)

        def ring_rdma(i, s, d, chunk0, nchunks, nbr):
            cols = pl.ds(chunk0 * NC, nchunks * NC)
            return pltpu.make_async_remote_copy(
                src_ref=comm_ref.at[(s - 1) % 2, d, :, cols],
                dst_ref=comm_ref.at[s % 2, d, :, cols],
                send_sem=hsend.at[i],
                recv_sem=hrecv.at[i],
                device_id=(nbr,),
                device_id_type=pl.DeviceIdType.MESH,
            )

        def hop_rdma(s, d, sub, nbr):
            i = 6 + (s - 2) * 4 + d * 2 + sub
            return ring_rdma(i, s, d, sub * SEED_SUB, SEED_SUB, nbr)

        QS = SEED_SUB // 2
        xcp0 = pltpu.make_async_copy(
            x_hbm.at[pl.ds(left * M_BLK, M_BLK), :], xbuf.at[0], xsems.at[0])
        xcp0.start()
        xcp1 = pltpu.make_async_copy(
            x_hbm.at[pl.ds(right * M_BLK, M_BLK), :], xbuf.at[1], xsems.at[1])
        xcp1.start()
        pltpu.make_async_copy(
            w_hbm.at[:, pl.ds(0, NC)], wbuf.at[0], wsems.at[0]).start()

        barrier_sem = pltpu.get_barrier_semaphore()
        for nbr in (left, right):
            pl.semaphore_signal(barrier_sem, inc=1, device_id=(nbr,),
                                device_id_type=pl.DeviceIdType.MESH)
        pl.semaphore_wait(barrier_sem, 2)
        xcp0.wait()
        xcp1.wait()

        pending = {}
        seed_plan = [
            (0, 0, QS, 0, 0), (1, 0, QS, 3, 0),
            (0, QS, QS, 1, 0), (1, QS, QS, 4, 0),
            (0, SEED_SUB, SEED_SUB, 2, 1), (1, SEED_SUB, SEED_SUB, 5, 1),
        ]
        for d, c0, nch, i, sub in seed_plan:
            j, nbr = (left, right) if d == 0 else (right, left)
            dir_partials(j, d, comm_ref.at[0], c0, c0 + nch,
                         load_x=False, skip_w_start=(i == 0))
            rd = ring_rdma(i, 1, d, c0, nch, nbr)
            rd.start()
            pending.setdefault((1, d, sub), []).append(rd)

        def add_range(d, recv_slot, c0, c1):
            def add_chunk(c, _):
                csl = pl.ds(c * NC, NC)
                acc = (comm_ref[recv_slot, d, :, csl].astype(jnp.float32)
                       + pacc_ref[d, :, csl].astype(jnp.float32))
                comm_ref[recv_slot, d, :, csl] = acc.astype(jnp.bfloat16)
                return 0
            lax.fori_loop(c0, c1, add_chunk, 0)

        def fin_range(d, recv_slot, c0, c1, am):
            def fin_chunk(c, am):
                csl = pl.ds(c * NC, NC)
                y = (comm_ref[recv_slot, d, :, csl].astype(jnp.float32)
                     + pacc_ref[d, :, csl].astype(jnp.float32))
                y = jnp.maximum(y, 0.0)
                comm_ref[recv_slot, d, :, csl] = y.astype(jnp.bfloat16)
                return jnp.maximum(am, jnp.max(y))
            return lax.fori_loop(c0, c1, fin_chunk, am)

        amax = jnp.float32(0.0)
        for s in range(1, N_DEV):
            recv_slot = s % 2
            jr = lax.rem(p + 2 * N_DEV - 1 - s, N_DEV)
            jl = lax.rem(p + 1 + s, N_DEV)
            dir_partials(jr, 0, pacc_ref)
            if s % 2 == 1:
                dir_partials(jl, 1, pacc_ref, load_x=False, xslot=0)
            else:
                dir_partials(jl, 1, pacc_ref)

            for sub in (0, 1):
                for d, nbr in ((0, right), (1, left)):
                    for rd in pending[(s, d, sub)]:
                        rd.wait()
                    if s < N_DEV - 1:
                        add_range(d, recv_slot,
                                  sub * SEED_SUB, (sub + 1) * SEED_SUB)
                        nxt = hop_rdma(s + 1, d, sub, nbr)
                        nxt.start()
                        pending[(s + 1, d, sub)] = [nxt]
                    else:
                        amax = fin_range(d, recv_slot, sub * SEED_SUB,
                                         (sub + 1) * SEED_SUB, amax)
        final_slot = (N_DEV - 1) % 2

        amax_ref[pl.ds(p, 1), :] = jnp.full((1, 128), amax, jnp.float32)
        sends = []
        for off in range(1, N_DEV):
            tgt = lax.rem(p + off, N_DEV)
            a = pltpu.make_async_remote_copy(
                src_ref=amax_ref.at[pl.ds(p, 1)],
                dst_ref=amax_ref.at[pl.ds(p, 1)],
                send_sem=a_send_sems.at[off - 1],
                recv_sem=a_recv_sems.at[off - 1],
                device_id=(tgt,),
                device_id_type=pl.DeviceIdType.MESH,
            )
            a.start()
            sends.append(a)
        for off in range(1, N_DEV):
            src = lax.rem(p + N_DEV - off, N_DEV)
            r = pltpu.make_async_remote_copy(
                src_ref=amax_ref.at[pl.ds(p, 1)],
                dst_ref=amax_ref.at[pl.ds(src, 1)],
                send_sem=a_send_sems.at[off - 1],
                recv_sem=a_recv_sems.at[off - 1],
                device_id=(src,),
                device_id_type=pl.DeviceIdType.MESH,
            )
            r.wait_recv()
        for a in sends:
            a.wait_send()

        amax_g = jnp.max(amax_ref[:, :])
        scale = amax_g / 127.0
        inv_scale = 127.0 / amax_g

        cps = []
        for c in range(2 * NCH):
            d = c // NCH
            b = c % 2
            csl = pl.ds((c % NCH) * NC, NC)
            if c >= 2:
                cps[c - 2].wait()
            y = comm_ref[final_slot, d, :, csl].astype(jnp.float32)
            q = jnp.clip(jnp.round(y * inv_scale), 0.0, 127.0)
            obuf[b] = (q * scale).astype(jnp.bfloat16)
            cp = pltpu.make_async_copy(
                obuf.at[b],
                out_hbm.at[:, pl.ds(d * HALF + (c % NCH) * NC, NC)],
                odma_sems.at[b])
            cp.start()
            cps.append(cp)
        cps[-2].wait()
        cps[-1].wait()

    return pl.pallas_call(
        body,
        out_shape=jax.ShapeDtypeStruct((M_BLK, N_TOT), jnp.bfloat16),
        in_specs=[
            pl.BlockSpec(memory_space=pl.ANY),
            pl.BlockSpec(memory_space=pl.ANY),
        ],
        out_specs=pl.BlockSpec(memory_space=pl.ANY),
        scratch_shapes=[
            pltpu.VMEM((2, 2, M_BLK, HALF), jnp.bfloat16),
            pltpu.VMEM((2, M_BLK, HALF), jnp.bfloat16),
            pltpu.VMEM((2, M_BLK, 1024), jnp.float32),
            pltpu.VMEM((2, 1024, NC), jnp.float32),
            pltpu.VMEM((2, M_BLK, NC), jnp.bfloat16),
            pltpu.VMEM((N_DEV, 128), jnp.float32),
            pltpu.SemaphoreType.DMA((14,)),
            pltpu.SemaphoreType.DMA((14,)),
            pltpu.SemaphoreType.DMA((N_DEV - 1,)),
            pltpu.SemaphoreType.DMA((N_DEV - 1,)),
            pltpu.SemaphoreType.DMA((2,)),
            pltpu.SemaphoreType.DMA((2,)),
            pltpu.SemaphoreType.DMA((2,)),
        ],
        compiler_params=pltpu.CompilerParams(
            collective_id=0,
            vmem_limit_bytes=67000000,
        ),
    )(x, w_mat)
